# baseline (speedup 1.0000x reference)
import jax
import jax.numpy as jnp
from jax import lax
from jax.experimental import pallas as pl
from jax.experimental.pallas import tpu as pltpu
from jax.experimental.pallas import tpu_sc as plsc

_L = 16
_N = 256
_NF = 255.0


def _sc_body(x_hbm, grid_hbm, out_hbm, buf_v, buf_o, sem):
    cpx = pltpu.async_copy(x_hbm, buf_v.at[pl.ds(8, 3)], sem)

    lane = lax.iota(jnp.int32, _L)
    bxl = (lane >> 3) & 1
    byl = (lane >> 2) & 1
    bzl = (lane >> 1) & 1
    chl = lane & 1
    cpx.wait()

    def bc(d):
        return plsc.load_gather(buf_v, [jnp.full((_L,), d + 8, jnp.int32)])

    def prep(c):
        f = c * _NF
        a0 = jnp.clip(jnp.clip(f, 0.0, _NF).astype(jnp.int32), 0, _N - 1)
        a1 = jnp.minimum(a0 + 1, _N - 1)
        w = f - a0.astype(jnp.float32)
        ok = (f >= 0.0) & (f <= _NF)
        return a0, a1, w, ok

    ix0, ix1, wx, okx = prep(bc(0))
    iy0, iy1, wy, oky = prep(bc(1))
    iz0, iz1, wz, okz = prep(bc(2))

    bx, by, bz, ch = bxl, byl, bzl, chl

    ixv = jnp.where(bx == 1, ix1, ix0)
    iyv = jnp.where(by == 1, iy1, iy0)
    izv = jnp.where(bz == 1, iz1, iz0)
    flat = (ixv * _N + iyv) * 512 + (izv >> 7) * 256 + ch * 128 + (izv & 127)

    cp = pltpu.async_copy(grid_hbm.at[flat], buf_v, sem)

    wxv = jnp.where(bx == 1, wx, 1.0 - wx)
    wyv = jnp.where(by == 1, wy, 1.0 - wy)
    wzv = jnp.where(bz == 1, wz, 1.0 - wz)
    validf = jnp.where(okx & oky & okz, 1.0, 0.0)
    wv = wxv * wyv * wzv * validf

    cp.wait()
    v = buf_v[...] * wv
    for stride in (8, 4, 2):
        buf_o[...] = v
        v = v + plsc.load_gather(buf_o, [lane ^ stride])
    buf_o[...] = v
    pltpu.sync_copy(buf_o.at[pl.ds(0, 2)], out_hbm)


_sc_interp = pl.kernel(
    _sc_body,
    out_type=jax.ShapeDtypeStruct((2,), jnp.float32),
    mesh=plsc.VectorSubcoreMesh(
        core_axis_name="c", subcore_axis_name="s", num_cores=1, num_subcores=1
    ),
    compiler_params=pltpu.CompilerParams(needs_layout_passes=False),
    scratch_types=[
        pltpu.VMEM((_L,), jnp.float32),
        pltpu.VMEM((_L,), jnp.float32),
        pltpu.SemaphoreType.DMA,
    ],
)


def kernel(x, grid):
    gridf = grid.reshape(_N, _N, 2, 128, 2).transpose(0, 1, 2, 4, 3).reshape(-1)
    return _sc_interp(x, gridf)


# speedup vs baseline: 1.0064x; 1.0064x over previous
import jax
import jax.numpy as jnp
from jax import lax
from jax.experimental import pallas as pl
from jax.experimental.pallas import tpu as pltpu
from jax.experimental.pallas import tpu_sc as plsc

_L = 16
_N = 256
_NF = 255.0


def _sc_body(x_hbm, grid_hbm, out_hbm, buf_v, buf_o, sem):
    cpx = pltpu.async_copy(x_hbm, buf_v.at[pl.ds(8, 3)], sem)

    lane = lax.iota(jnp.int32, _L)
    bxl = (lane >> 3) & 1
    byl = (lane >> 2) & 1
    bzl = (lane >> 1) & 1
    chl = lane & 1
    buf_o[...] = jnp.zeros((_L,), jnp.float32)
    cpx.wait()

    def bc(d):
        return plsc.load_gather(buf_v, [jnp.full((_L,), d + 8, jnp.int32)])

    def prep(c):
        f = c * _NF
        a0 = jnp.clip(jnp.clip(f, 0.0, _NF).astype(jnp.int32), 0, _N - 1)
        a1 = jnp.minimum(a0 + 1, _N - 1)
        w = f - a0.astype(jnp.float32)
        ok = (f >= 0.0) & (f <= _NF)
        return a0, a1, w, ok

    ix0, ix1, wx, okx = prep(bc(0))
    iy0, iy1, wy, oky = prep(bc(1))
    iz0, iz1, wz, okz = prep(bc(2))

    bx, by, bz, ch = bxl, byl, bzl, chl

    ixv = jnp.where(bx == 1, ix1, ix0)
    iyv = jnp.where(by == 1, iy1, iy0)
    izv = jnp.where(bz == 1, iz1, iz0)
    flat = (ixv * _N + iyv) * 512 + (izv >> 7) * 256 + ch * 128 + (izv & 127)

    cp = pltpu.async_copy(grid_hbm.at[flat], buf_v, sem)

    wxv = jnp.where(bx == 1, wx, 1.0 - wx)
    wyv = jnp.where(by == 1, wy, 1.0 - wy)
    wzv = jnp.where(bz == 1, wz, 1.0 - wz)
    validf = jnp.where(okx & oky & okz, 1.0, 0.0)
    wv = wxv * wyv * wzv * validf

    cp.wait()
    plsc.addupdate_scatter(buf_o, [ch], buf_v[...] * wv)
    pltpu.sync_copy(buf_o.at[pl.ds(0, 2)], out_hbm)


_sc_interp = pl.kernel(
    _sc_body,
    out_type=jax.ShapeDtypeStruct((2,), jnp.float32),
    mesh=plsc.VectorSubcoreMesh(
        core_axis_name="c", subcore_axis_name="s", num_cores=1, num_subcores=1
    ),
    compiler_params=pltpu.CompilerParams(needs_layout_passes=False),
    scratch_types=[
        pltpu.VMEM((_L,), jnp.float32),
        pltpu.VMEM((_L,), jnp.float32),
        pltpu.SemaphoreType.DMA,
    ],
)


def kernel(x, grid):
    gridf = grid.reshape(_N, _N, 2, 128, 2).transpose(0, 1, 2, 4, 3).reshape(-1)
    return _sc_interp(x, gridf)
